# baseline (speedup 1.0000x reference)
import dataclasses
import functools

import jax
import jax.numpy as jnp
from jax import lax
from jax.experimental import pallas as pl
from jax.experimental.pallas import tpu as pltpu
from jax.experimental.pallas import tpu_sc as plsc

_NUM_CORES = 2
_NUM_SUBCORES = 16
_NUM_WORKERS = _NUM_CORES * _NUM_SUBCORES
_CHB = 1024


def kernel(token_ids, weight):
    B, S = token_ids.shape
    D = weight.shape[1]
    n_chunks = (B // _CHB) * S
    per_w = n_chunks // _NUM_WORKERS
    assert B % _CHB == 0 and n_chunks % _NUM_WORKERS == 0
    blocks_per_s = B // _CHB

    tids_t = token_ids.T
    mesh = plsc.VectorSubcoreMesh(core_axis_name="c", subcore_axis_name="s")

    @functools.partial(
        pl.kernel,
        mesh=mesh,
        out_type=jax.ShapeDtypeStruct((S, D, B), weight.dtype),
        compiler_params=dataclasses.replace(
            pltpu.CompilerParams(use_tc_tiling_on_sc=False),
            needs_layout_passes=False,
        ),
        scratch_types=[
            pltpu.VMEM((1, _CHB), jnp.int32),
            pltpu.VMEM((_CHB, D), jnp.float32),
            pltpu.VMEM((D, _CHB), jnp.float32),
            pltpu.SemaphoreType.DMA,
        ],
    )
    def gather_kernel(w_hbm, idx_hbm, out_hbm, idx_v, rows_v, outt_v, sem):
        wid = lax.axis_index("s") * _NUM_CORES + lax.axis_index("c")
        row_base = lax.iota(jnp.int32, 16)
        cols = [jnp.full((16,), d, jnp.int32) for d in range(D)]

        @pl.loop(0, per_w)
        def _(k):
            g = wid * per_w + k
            s = g // blocks_per_s
            b0 = (g % blocks_per_s) * _CHB
            pltpu.sync_copy(idx_hbm.at[pl.ds(s, 1), pl.ds(b0, _CHB)], idx_v)
            pltpu.async_copy(w_hbm.at[idx_v.at[0]], rows_v, sem).wait()

            @pl.loop(0, _CHB, step=16)
            def _(j0):
                row = row_base + j0
                for d in range(D):
                    v = plsc.load_gather(rows_v, [row, cols[d]])
                    outt_v[d, pl.ds(j0, 16)] = v

            pltpu.sync_copy(outt_v, out_hbm.at[s, :, pl.ds(b0, _CHB)])

    out = gather_kernel(weight, tids_t)
    return out.transpose(2, 0, 1)


# speedup vs baseline: 1.4814x; 1.4814x over previous
import dataclasses
import functools

import jax
import jax.numpy as jnp
from jax import lax
from jax.experimental import pallas as pl
from jax.experimental.pallas import tpu as pltpu
from jax.experimental.pallas import tpu_sc as plsc

_NUM_CORES = 2
_NUM_SUBCORES = 16
_NUM_WORKERS = _NUM_CORES * _NUM_SUBCORES
_CHB = 1024


def kernel(token_ids, weight):
    B, S = token_ids.shape
    D = weight.shape[1]
    n_chunks = (B // _CHB) * S
    per_w = n_chunks // _NUM_WORKERS
    assert B % _CHB == 0 and n_chunks % _NUM_WORKERS == 0
    blocks_per_s = B // _CHB

    tids_t = token_ids.T
    mesh = plsc.VectorSubcoreMesh(core_axis_name="c", subcore_axis_name="s")

    @functools.partial(
        pl.kernel,
        mesh=mesh,
        out_type=jax.ShapeDtypeStruct((S, D, B), weight.dtype),
        compiler_params=dataclasses.replace(
            pltpu.CompilerParams(use_tc_tiling_on_sc=False),
            needs_layout_passes=False,
        ),
        scratch_types=[
            pltpu.VMEM((1, _CHB), jnp.int32),
            pltpu.VMEM((_CHB, D), jnp.float32),
            pltpu.VMEM((D, _CHB + 1), jnp.float32),
            pltpu.SemaphoreType.DMA,
        ],
    )
    def gather_kernel(w_hbm, idx_hbm, out_hbm, idx_v, rows_v, outt_v, sem):
        wid = lax.axis_index("s") * _NUM_CORES + lax.axis_index("c")
        rows = [lax.iota(jnp.int32, 16) + 16 * h for h in range(D // 16)]

        @pl.loop(0, per_w)
        def _(k):
            g = wid * per_w + k
            s = g // blocks_per_s
            b0 = (g % blocks_per_s) * _CHB
            pltpu.sync_copy(idx_hbm.at[pl.ds(s, 1), pl.ds(b0, _CHB)], idx_v)
            pltpu.async_copy(w_hbm.at[idx_v.at[0]], rows_v, sem).wait()

            @pl.loop(0, _CHB, step=4)
            def _(j0):
                for u in range(4):
                    j = j0 + u
                    col = jnp.full((16,), j, jnp.int32)
                    for h in range(D // 16):
                        v = rows_v[j, pl.ds(16 * h, 16)]
                        plsc.store_scatter(outt_v, [rows[h], col], v)

            pltpu.sync_copy(
                outt_v.at[:, pl.ds(0, _CHB)], out_hbm.at[s, :, pl.ds(b0, _CHB)]
            )

    out = gather_kernel(weight, tids_t)
    return out.transpose(2, 0, 1)


# speedup vs baseline: 1.5784x; 1.0655x over previous
import dataclasses
import functools

import jax
import jax.numpy as jnp
from jax import lax
from jax.experimental import pallas as pl
from jax.experimental.pallas import tpu as pltpu
from jax.experimental.pallas import tpu_sc as plsc

_NUM_CORES = 2
_NUM_SUBCORES = 16
_NUM_WORKERS = _NUM_CORES * _NUM_SUBCORES
_CHB = 512


def kernel(token_ids, weight):
    B, S = token_ids.shape
    D = weight.shape[1]
    n_chunks = (B // _CHB) * S
    per_w = n_chunks // _NUM_WORKERS
    assert B % _CHB == 0 and n_chunks % _NUM_WORKERS == 0 and per_w % 2 == 0
    blocks_per_s = B // _CHB

    tids_t = token_ids.T
    mesh = plsc.VectorSubcoreMesh(core_axis_name="c", subcore_axis_name="s")

    @functools.partial(
        pl.kernel,
        mesh=mesh,
        out_type=jax.ShapeDtypeStruct((S, D, B), weight.dtype),
        compiler_params=dataclasses.replace(
            pltpu.CompilerParams(use_tc_tiling_on_sc=False),
            needs_layout_passes=False,
        ),
        scratch_types=[
            pltpu.VMEM((1, _CHB), jnp.int32),
            pltpu.VMEM((1, _CHB), jnp.int32),
            pltpu.VMEM((_CHB, D), jnp.float32),
            pltpu.VMEM((_CHB, D), jnp.float32),
            pltpu.VMEM((D, _CHB + 1), jnp.float32),
            pltpu.VMEM((D, _CHB + 1), jnp.float32),
            pltpu.SemaphoreType.DMA,
            pltpu.SemaphoreType.DMA,
            pltpu.SemaphoreType.DMA,
            pltpu.SemaphoreType.DMA,
        ],
    )
    def gather_kernel(
        w_hbm, idx_hbm, out_hbm,
        idx0, idx1, rows0, rows1, outt0, outt1, sg0, sg1, so0, so1,
    ):
        wid = lax.axis_index("s") * _NUM_CORES + lax.axis_index("c")
        base = wid * per_w
        idx_v = (idx0, idx1)
        rows_v = (rows0, rows1)
        outt_v = (outt0, outt1)
        sem_g = (sg0, sg1)
        sem_o = (so0, so1)
        rows16 = [lax.iota(jnp.int32, 16) + 16 * h for h in range(D // 16)]

        def chunk_pos(k):
            g = base + k
            return g // blocks_per_s, (g % blocks_per_s) * _CHB

        def launch(k, b):
            s, b0 = chunk_pos(k)
            pltpu.sync_copy(idx_hbm.at[pl.ds(s, 1), pl.ds(b0, _CHB)], idx_v[b])
            pltpu.async_copy(w_hbm.at[idx_v[b].at[0]], rows_v[b], sem_g[b])

        def drain_gather(b):
            pltpu.make_async_copy(
                w_hbm.at[idx_v[b].at[0]], rows_v[b], sem_g[b]
            ).wait()

        def drain_out(k, b):
            s, b0 = chunk_pos(k)
            pltpu.make_async_copy(
                outt_v[b].at[:, pl.ds(0, _CHB)],
                out_hbm.at[s, :, pl.ds(b0, _CHB)],
                sem_o[b],
            ).wait()

        def transpose_and_store(k, b):
            rv, ov = rows_v[b], outt_v[b]

            @pl.loop(0, _CHB, step=4)
            def _(j0):
                for u in range(4):
                    j = j0 + u
                    col = jnp.full((16,), j, jnp.int32)
                    for h in range(D // 16):
                        v = rv[j, pl.ds(16 * h, 16)]
                        plsc.store_scatter(ov, [rows16[h], col], v)

            s, b0 = chunk_pos(k)
            pltpu.async_copy(
                ov.at[:, pl.ds(0, _CHB)],
                out_hbm.at[s, :, pl.ds(b0, _CHB)],
                sem_o[b],
            )

        launch(0, 0)
        launch(1, 1)
        drain_gather(0)
        transpose_and_store(0, 0)
        drain_gather(1)
        launch(2, 0)
        transpose_and_store(1, 1)

        @pl.loop(2, per_w - 2, step=2)
        def _(k0):
            for b in range(2):
                k = k0 + b
                drain_gather(b)
                launch(k + 1, 1 - b)
                drain_out(k - 2, b)
                transpose_and_store(k, b)

        drain_gather(0)
        launch(per_w - 1, 1)
        drain_out(per_w - 4, 0)
        transpose_and_store(per_w - 2, 0)
        drain_gather(1)
        drain_out(per_w - 3, 1)
        transpose_and_store(per_w - 1, 1)
        drain_out(per_w - 2, 0)
        drain_out(per_w - 1, 1)

    out = gather_kernel(weight, tids_t)
    return out.transpose(2, 0, 1)
